# Initial kernel scaffold; baseline (speedup 1.0000x reference)
import functools

import jax
import jax.numpy as jnp
import numpy as np
from jax import lax
from jax.experimental import pallas as pl
from jax.experimental.pallas import tpu as pltpu
from jax.experimental.pallas import tpu_sc as plsc

KH, KW, PAD = 16, 16, 10
H, W, C = 128, 128, 3
D = C * KH * KW
K1, K2 = 2048, 1024
OH = H + 2 * PAD - KH + 1
OW = W + 2 * PAD - KW + 1
L = OH * OW
L_PAD = 17920
BLK_A = 512

NC, NS = 2, 16
NW = NC * NS
BPW = L_PAD // NW
CHUNK = 56
NCHUNK = BPW // CHUNK


def _nn_body(q_ref, k_ref, m_ref, ids_ref):
    q = q_ref[...]
    k = k_ref[...]
    dot = lax.dot_general(q, k, (((1,), (1,)), ((), ())),
                          preferred_element_type=jnp.float32)
    qsq = jnp.sum(q * q, axis=1, keepdims=True)
    ksq = jnp.sum(k * k, axis=1)[None, :]
    d2 = qsq - 2.0 * dot + ksq
    mn = jnp.min(d2, axis=1, keepdims=True)
    iota = lax.broadcasted_iota(jnp.int32, (BLK_A, K1), 1)
    ids = jnp.min(jnp.where(d2 == mn, iota, K1), axis=1)
    onehot = (iota == ids[:, None]).astype(jnp.float32)
    pat = lax.dot_general(onehot, m_ref[...], (((1,), (0,)), ((), ())),
                          preferred_element_type=jnp.float32)
    ids_ref[0, 0, :] = pat[:, 0].astype(jnp.int32)


def _nearest_pattern_ids(patches, keys1, mapping_f32):
    grid = L_PAD // BLK_A
    out = pl.pallas_call(
        _nn_body,
        grid=(grid,),
        in_specs=[
            pl.BlockSpec((BLK_A, D), lambda m: (m, 0)),
            pl.BlockSpec((K1, D), lambda m: (0, 0)),
            pl.BlockSpec((K1, 1), lambda m: (0, 0)),
        ],
        out_specs=pl.BlockSpec((1, 1, BLK_A), lambda m: (m, 0, 0)),
        out_shape=jax.ShapeDtypeStruct((grid, 1, BLK_A), jnp.int32),
    )(patches, keys1, mapping_f32)
    return out.reshape(L_PAD)


def _sc_gather(pat_ids, table):
    mesh = plsc.VectorSubcoreMesh(core_axis_name="c", subcore_axis_name="s")

    @functools.partial(
        pl.kernel,
        mesh=mesh,
        out_type=jax.ShapeDtypeStruct((L_PAD, D), jnp.float32),
        scratch_types=[
            pltpu.VMEM((BPW,), jnp.int32),
            pltpu.VMEM((CHUNK, D), jnp.float32),
            pltpu.SemaphoreType.DMA,
        ],
    )
    def k(ids_hbm, tab_hbm, out_hbm, idx_v, rows_v, sem):
        wid = lax.axis_index("s") * NC + lax.axis_index("c")
        base = wid * BPW
        pltpu.sync_copy(ids_hbm.at[pl.ds(base, BPW)], idx_v)

        def body(ci, carry):
            off = ci * CHUNK
            pltpu.async_copy(tab_hbm.at[idx_v.at[pl.ds(off, CHUNK)]],
                             rows_v, sem).wait()
            pltpu.sync_copy(rows_v, out_hbm.at[pl.ds(base + off, CHUNK)])
            return carry

        lax.fori_loop(0, NCHUNK, body, 0)

    return k(pat_ids, table)


def _fold_matrix():
    s = np.zeros((OW * KW, W + 2 * PAD), np.float32)
    for px in range(OW):
        for j in range(KW):
            s[px * KW + j, px + j] = 1.0
    return jnp.asarray(s)


def _fold_body(g_ref, s_ref, out_ref, acc_ref):
    py = pl.program_id(0)
    g = g_ref[0]
    gp = g.reshape(OW * KW, C * KH)
    r = lax.dot_general(gp, s_ref[...], (((0,), (0,)), ((), ())),
                        preferred_element_type=jnp.float32)
    acc_ref[py] = r

    @pl.when(py == OH - 1)
    def _():
        out_ref[...] = jnp.zeros((C, H, W), jnp.float32)
        for c in range(C):
            for i in range(KH):
                ylo = max(0, i - PAD)
                yhi = min(H - 1, H - KH + PAD + i)
                n = yhi - ylo + 1
                plo = ylo + PAD - i
                out_ref[c, ylo:ylo + n, :] += (
                    acc_ref[plo:plo + n, c * KH + i, PAD:PAD + W])
        m = jnp.max(out_ref[...])
        out_ref[...] = out_ref[...] / m


def _fold_normalize(g3, s_mat):
    return pl.pallas_call(
        _fold_body,
        grid=(OH,),
        in_specs=[
            pl.BlockSpec((1, OW, D), lambda py: (py, 0, 0)),
            pl.BlockSpec((OW * KW, W + 2 * PAD), lambda py: (0, 0)),
        ],
        out_specs=pl.BlockSpec((C, H, W), lambda py: (0, 0, 0)),
        out_shape=jax.ShapeDtypeStruct((C, H, W), jnp.float32),
        scratch_shapes=[pltpu.VMEM((OH, C * KH, W + 2 * PAD), jnp.float32)],
    )(g3, s_mat)


def kernel(image, keys1, values2, mapping_table):
    img = jnp.transpose(image, (2, 0, 1))
    padded = jnp.pad(img, ((0, 0), (PAD, PAD), (PAD, PAD)))
    ri = jnp.arange(OH)[:, None] + jnp.arange(KH)[None, :]
    ci = jnp.arange(OW)[:, None] + jnp.arange(KW)[None, :]
    patches = padded[:, ri[:, None, :, None], ci[None, :, None, :]]
    patches = jnp.transpose(patches, (1, 2, 0, 3, 4)).reshape(L, D)
    patches = jnp.pad(patches, ((0, L_PAD - L), (0, 0)))

    mapping_f32 = mapping_table.astype(jnp.float32).reshape(K1, 1)
    pat_ids = _nearest_pattern_ids(patches, keys1, mapping_f32)

    v3 = values2.reshape(K2, C * KH, KW).swapaxes(1, 2).reshape(K2, D)
    g = _sc_gather(pat_ids, v3)

    g3 = g[:L].reshape(OH, OW, D)
    out_chw = _fold_normalize(g3, _fold_matrix())
    return jnp.transpose(out_chw, (1, 2, 0))


# speedup vs baseline: 1.3753x; 1.3753x over previous
import functools

import jax
import jax.numpy as jnp
import numpy as np
from jax import lax
from jax.experimental import pallas as pl
from jax.experimental.pallas import tpu as pltpu
from jax.experimental.pallas import tpu_sc as plsc

KH, KW, PAD = 16, 16, 10
H, W, C = 128, 128, 3
D = C * KH * KW
K1, K2 = 2048, 1024
OH = H + 2 * PAD - KH + 1
OW = W + 2 * PAD - KW + 1
L = OH * OW
L_PAD = 17920
BLK_A = 512

NC, NS = 2, 16
NW = NC * NS
BPW = L_PAD // NW
CHUNK = 56
NCHUNK = BPW // CHUNK


def _nn_body(q_ref, k_ref, m_ref, ids_ref):
    q = q_ref[...]
    k = k_ref[...]
    dot = lax.dot_general(q, k, (((1,), (1,)), ((), ())),
                          preferred_element_type=jnp.float32)
    qsq = jnp.sum(q * q, axis=1, keepdims=True)
    ksq = jnp.sum(k * k, axis=1)[None, :]
    d2 = qsq - 2.0 * dot + ksq
    mn = jnp.min(d2, axis=1, keepdims=True)
    iota = lax.broadcasted_iota(jnp.int32, (BLK_A, K1), 1)
    ids = jnp.min(jnp.where(d2 == mn, iota, K1), axis=1)
    pat = jnp.sum(jnp.where(iota == ids[:, None], m_ref[...], 0), axis=1)
    ids_ref[0, 0, :] = pat


def _nearest_pattern_ids(patches, keys1, mapping_f32):
    grid = L_PAD // BLK_A
    out = pl.pallas_call(
        _nn_body,
        grid=(grid,),
        in_specs=[
            pl.BlockSpec((BLK_A, D), lambda m: (m, 0)),
            pl.BlockSpec((K1, D), lambda m: (0, 0)),
            pl.BlockSpec((1, K1), lambda m: (0, 0)),
        ],
        out_specs=pl.BlockSpec((1, 1, BLK_A), lambda m: (m, 0, 0)),
        out_shape=jax.ShapeDtypeStruct((grid, 1, BLK_A), jnp.int32),
    )(patches, keys1, mapping_f32)
    return out.reshape(L_PAD)


def _sc_gather(pat_ids, table):
    mesh = plsc.VectorSubcoreMesh(core_axis_name="c", subcore_axis_name="s")
    ids3 = pat_ids.reshape(NW, NCHUNK, CHUNK)

    @functools.partial(
        pl.kernel,
        mesh=mesh,
        out_type=jax.ShapeDtypeStruct((NW, NCHUNK, CHUNK, D), jnp.float32),
        scratch_types=[
            pltpu.VMEM((NCHUNK, CHUNK), jnp.int32),
            pltpu.VMEM((CHUNK, D), jnp.float32),
            pltpu.SemaphoreType.DMA,
        ],
    )
    def k(ids_hbm, tab_hbm, out_hbm, idx_v, rows_v, sem):
        wid = lax.axis_index("s") * NC + lax.axis_index("c")
        pltpu.sync_copy(ids_hbm.at[wid], idx_v)

        def body(ci, carry):
            pltpu.async_copy(tab_hbm.at[idx_v.at[ci]], rows_v, sem).wait()
            pltpu.sync_copy(rows_v, out_hbm.at[wid, ci])
            return carry

        lax.fori_loop(0, NCHUNK, body, 0)

    return k(ids3, table).reshape(L_PAD, D)


def _fold_matrix():
    s = np.zeros((OW * KW, W + 2 * PAD), np.float32)
    for px in range(OW):
        for j in range(KW):
            s[px * KW + j, px + j] = 1.0
    return jnp.asarray(s)


def _fold_body(g_ref, s_ref, out_ref, acc_ref):
    py = pl.program_id(0)
    gp = g_ref[0]
    r = lax.dot_general(gp, s_ref[...], (((0,), (0,)), ((), ())),
                        preferred_element_type=jnp.float32)
    acc_ref[py] = r

    @pl.when(py == OH - 1)
    def _():
        out_ref[...] = jnp.zeros((C, H, W), jnp.float32)
        for c in range(C):
            for i in range(KH):
                ylo = max(0, i - PAD)
                yhi = min(H - 1, H - KH + PAD + i)
                n = yhi - ylo + 1
                plo = ylo + PAD - i
                out_ref[c, ylo:ylo + n, :] += (
                    acc_ref[plo:plo + n, c * KH + i, PAD:PAD + W])
        m = jnp.max(out_ref[...])
        out_ref[...] = out_ref[...] / m


def _fold_normalize(g3, s_mat):
    return pl.pallas_call(
        _fold_body,
        grid=(OH,),
        in_specs=[
            pl.BlockSpec((1, OW * KW, C * KH), lambda py: (py, 0, 0)),
            pl.BlockSpec((OW * KW, W + 2 * PAD), lambda py: (0, 0)),
        ],
        out_specs=pl.BlockSpec((C, H, W), lambda py: (0, 0, 0)),
        out_shape=jax.ShapeDtypeStruct((C, H, W), jnp.float32),
        scratch_shapes=[pltpu.VMEM((OH, C * KH, W + 2 * PAD), jnp.float32)],
    )(g3, s_mat)


def kernel(image, keys1, values2, mapping_table):
    img = jnp.transpose(image, (2, 0, 1))
    padded = jnp.pad(img, ((0, 0), (PAD, PAD), (PAD, PAD)))
    ri = jnp.arange(OH)[:, None] + jnp.arange(KH)[None, :]
    ci = jnp.arange(OW)[:, None] + jnp.arange(KW)[None, :]
    patches = padded[:, ri[:, None, :, None], ci[None, :, None, :]]
    patches = jnp.transpose(patches, (1, 2, 0, 3, 4)).reshape(L, D)
    patches = jnp.pad(patches, ((0, L_PAD - L), (0, 0)))

    pat_ids = _nearest_pattern_ids(patches, keys1,
                                   mapping_table.reshape(1, K1))

    v3 = values2.reshape(K2, C * KH, KW).swapaxes(1, 2).reshape(K2, D)
    g = _sc_gather(pat_ids, v3)

    g3 = g[:L].reshape(OH, OW * KW, C * KH)
    out_chw = _fold_normalize(g3, _fold_matrix())
    return jnp.transpose(out_chw, (1, 2, 0))


# speedup vs baseline: 44.3352x; 32.2356x over previous
import functools

import jax
import jax.numpy as jnp
import numpy as np
from jax import lax
from jax.experimental import pallas as pl
from jax.experimental.pallas import tpu as pltpu
from jax.experimental.pallas import tpu_sc as plsc

KH, KW, PAD = 16, 16, 10
H, W, C = 128, 128, 3
D = C * KH * KW
K1, K2 = 2048, 1024
OH = H + 2 * PAD - KH + 1
OW = W + 2 * PAD - KW + 1
L = OH * OW
L_PAD = 17920
BLK_A = 512

NC, NS = 2, 16
NW = NC * NS
BPW = L_PAD // NW
CHUNK = 56
NCHUNK = BPW // CHUNK


def _nn_body(q_ref, k_ref, m_ref, ids_ref):
    q = q_ref[...]
    k = k_ref[...]
    dot = lax.dot_general(q, k, (((1,), (1,)), ((), ())),
                          preferred_element_type=jnp.float32)
    qsq = jnp.sum(q * q, axis=1, keepdims=True)
    ksq = jnp.sum(k * k, axis=1)[None, :]
    d2 = qsq - 2.0 * dot + ksq
    mn = jnp.min(d2, axis=1, keepdims=True)
    iota = lax.broadcasted_iota(jnp.int32, (BLK_A, K1), 1)
    ids = jnp.min(jnp.where(d2 == mn, iota, K1), axis=1)
    pat = jnp.sum(jnp.where(iota == ids[:, None], m_ref[...], 0), axis=1)
    ids_ref[0, 0, :] = pat


def _nearest_pattern_ids(patches, keys1, mapping_f32):
    grid = L_PAD // BLK_A
    out = pl.pallas_call(
        _nn_body,
        grid=(grid,),
        in_specs=[
            pl.BlockSpec((BLK_A, D), lambda m: (m, 0)),
            pl.BlockSpec((K1, D), lambda m: (0, 0)),
            pl.BlockSpec((1, K1), lambda m: (0, 0)),
        ],
        out_specs=pl.BlockSpec((1, 1, BLK_A), lambda m: (m, 0, 0)),
        out_shape=jax.ShapeDtypeStruct((grid, 1, BLK_A), jnp.int32),
    )(patches, keys1, mapping_f32)
    return out.reshape(L_PAD)


def _sc_gather(pat_ids, table):
    mesh = plsc.VectorSubcoreMesh(core_axis_name="c", subcore_axis_name="s")
    ids3 = pat_ids.reshape(NW, NCHUNK, CHUNK)

    @functools.partial(
        pl.kernel,
        mesh=mesh,
        out_type=jax.ShapeDtypeStruct((NW, NCHUNK, CHUNK, D), jnp.float32),
        scratch_types=[
            pltpu.VMEM((NCHUNK, CHUNK), jnp.int32),
            pltpu.VMEM((CHUNK, D), jnp.float32),
            pltpu.SemaphoreType.DMA,
        ],
    )
    def k(ids_hbm, tab_hbm, out_hbm, idx_v, rows_v, sem):
        wid = lax.axis_index("s") * NC + lax.axis_index("c")
        pltpu.sync_copy(ids_hbm.at[wid], idx_v)

        def body(ci, carry):
            pltpu.async_copy(tab_hbm.at[idx_v.at[ci]], rows_v, sem).wait()
            pltpu.sync_copy(rows_v, out_hbm.at[wid, ci])
            return carry

        lax.fori_loop(0, NCHUNK, body, 0)

    return k(ids3, table).reshape(L_PAD, D)


def _fold_matrix():
    s = np.zeros((OW * KW, W + 2 * PAD), np.float32)
    for px in range(OW):
        for j in range(KW):
            s[px * KW + j, px + j] = 1.0
    return jnp.asarray(s)


def _fold_body(g_ref, s_ref, out_ref, acc_ref):
    py = pl.program_id(0)
    gp = g_ref[0]
    r = lax.dot_general(gp, s_ref[...], (((0,), (0,)), ((), ())),
                        preferred_element_type=jnp.float32)
    acc_ref[py] = r

    @pl.when(py == OH - 1)
    def _():
        out_ref[...] = jnp.zeros((C, H, W), jnp.float32)
        for c in range(C):
            for i in range(KH):
                ylo = max(0, i - PAD)
                yhi = min(H - 1, H - KH + PAD + i)
                n = yhi - ylo + 1
                plo = ylo + PAD - i
                out_ref[c, ylo:ylo + n, :] += (
                    acc_ref[plo:plo + n, c * KH + i, PAD:PAD + W])
        m = jnp.max(out_ref[...])
        out_ref[...] = out_ref[...] / m


def _fold_normalize(g3, s_mat):
    return pl.pallas_call(
        _fold_body,
        grid=(OH,),
        in_specs=[
            pl.BlockSpec((1, OW * KW, C * KH), lambda py: (py, 0, 0)),
            pl.BlockSpec((OW * KW, W + 2 * PAD), lambda py: (0, 0)),
        ],
        out_specs=pl.BlockSpec((C, H, W), lambda py: (0, 0, 0)),
        out_shape=jax.ShapeDtypeStruct((C, H, W), jnp.float32),
        scratch_shapes=[pltpu.VMEM((OH, C * KH, W + 2 * PAD), jnp.float32)],
    )(g3, s_mat)


def kernel(image, keys1, values2, mapping_table):
    img = jnp.transpose(image, (2, 0, 1))
    padded = jnp.pad(img, ((0, 0), (PAD, PAD), (PAD, PAD)))
    shards = jnp.stack(
        [padded[:, i:i + OH, j:j + OW] for i in range(KH) for j in range(KW)],
        axis=0)
    patches = jnp.transpose(shards, (2, 3, 1, 0)).reshape(L, D)
    patches = jnp.pad(patches, ((0, L_PAD - L), (0, 0)))

    pat_ids = _nearest_pattern_ids(patches, keys1,
                                   mapping_table.reshape(1, K1))

    v3 = values2.reshape(K2, C * KH, KW).swapaxes(1, 2).reshape(K2, D)
    g = _sc_gather(pat_ids, v3)

    g3 = g[:L].reshape(OH, OW * KW, C * KH)
    out_chw = _fold_normalize(g3, _fold_matrix())
    return jnp.transpose(out_chw, (1, 2, 0))


# speedup vs baseline: 70.3014x; 1.5857x over previous
import functools

import jax
import jax.numpy as jnp
import numpy as np
from jax import lax
from jax.experimental import pallas as pl
from jax.experimental.pallas import tpu as pltpu
from jax.experimental.pallas import tpu_sc as plsc

KH, KW, PAD = 16, 16, 10
H, W, C = 128, 128, 3
D = C * KH * KW
K1, K2 = 2048, 1024
OH = H + 2 * PAD - KH + 1
OW = W + 2 * PAD - KW + 1
L = OH * OW
L_PAD = 17920
BLK_A = 512

NC, NS = 2, 16
NW = NC * NS
BPW = L_PAD // NW
CHUNK = 56
NCHUNK = BPW // CHUNK


def _nn_body(q_ref, k_ref, m_ref, ids_ref):
    q = q_ref[...]
    k = k_ref[...]
    dot = lax.dot_general(k, q, (((1,), (0,)), ((), ())),
                          preferred_element_type=jnp.float32)
    qsq = jnp.sum(q * q, axis=0, keepdims=True)
    ksq = jnp.sum(k * k, axis=1, keepdims=True)
    d2 = qsq - 2.0 * dot + ksq
    mn = jnp.min(d2, axis=0, keepdims=True)
    iota = lax.broadcasted_iota(jnp.int32, (K1, BLK_A), 0)
    ids = jnp.min(jnp.where(d2 == mn, iota, K1), axis=0)
    pat = jnp.sum(jnp.where(iota == ids[None, :], m_ref[...], 0), axis=0)
    ids_ref[0, 0, :] = pat


def _nearest_pattern_ids(patches_t, keys1r, mapping_col):
    grid = L_PAD // BLK_A
    out = pl.pallas_call(
        _nn_body,
        grid=(grid,),
        in_specs=[
            pl.BlockSpec((D, BLK_A), lambda m: (0, m)),
            pl.BlockSpec((K1, D), lambda m: (0, 0)),
            pl.BlockSpec((K1, 1), lambda m: (0, 0)),
        ],
        out_specs=pl.BlockSpec((1, 1, BLK_A), lambda m: (m, 0, 0)),
        out_shape=jax.ShapeDtypeStruct((grid, 1, BLK_A), jnp.int32),
    )(patches_t, keys1r, mapping_col)
    return out.reshape(L_PAD)


def _sc_gather(pat_ids, table):
    mesh = plsc.VectorSubcoreMesh(core_axis_name="c", subcore_axis_name="s")
    ids3 = pat_ids.reshape(NW, NCHUNK, CHUNK)

    @functools.partial(
        pl.kernel,
        mesh=mesh,
        out_type=jax.ShapeDtypeStruct((NW, NCHUNK, CHUNK, D), jnp.float32),
        scratch_types=[
            pltpu.VMEM((NCHUNK, CHUNK), jnp.int32),
            pltpu.VMEM((CHUNK, D), jnp.float32),
            pltpu.SemaphoreType.DMA,
        ],
    )
    def k(ids_hbm, tab_hbm, out_hbm, idx_v, rows_v, sem):
        wid = lax.axis_index("s") * NC + lax.axis_index("c")
        pltpu.sync_copy(ids_hbm.at[wid], idx_v)

        def body(ci, carry):
            pltpu.async_copy(tab_hbm.at[idx_v.at[ci]], rows_v, sem).wait()
            pltpu.sync_copy(rows_v, out_hbm.at[wid, ci])
            return carry

        lax.fori_loop(0, NCHUNK, body, 0)

    return k(ids3, table).reshape(L_PAD, D)


def _fold_matrix():
    s = np.zeros((OW * KW, W + 2 * PAD), np.float32)
    for px in range(OW):
        for j in range(KW):
            s[px * KW + j, px + j] = 1.0
    return jnp.asarray(s)


def _fold_body(g_ref, s_ref, out_ref, acc_ref):
    py = pl.program_id(0)
    gp = g_ref[0]
    r = lax.dot_general(gp, s_ref[...], (((0,), (0,)), ((), ())),
                        preferred_element_type=jnp.float32)
    acc_ref[py] = r

    @pl.when(py == OH - 1)
    def _():
        out_ref[...] = jnp.zeros((C, H, W), jnp.float32)
        for c in range(C):
            for i in range(KH):
                ylo = max(0, i - PAD)
                yhi = min(H - 1, H - KH + PAD + i)
                n = yhi - ylo + 1
                plo = ylo + PAD - i
                out_ref[c, ylo:ylo + n, :] += (
                    acc_ref[plo:plo + n, c * KH + i, PAD:PAD + W])
        m = jnp.max(out_ref[...])
        out_ref[...] = out_ref[...] / m


def _fold_normalize(g3, s_mat):
    return pl.pallas_call(
        _fold_body,
        grid=(OH,),
        in_specs=[
            pl.BlockSpec((1, OW * KW, C * KH), lambda py: (py, 0, 0)),
            pl.BlockSpec((OW * KW, W + 2 * PAD), lambda py: (0, 0)),
        ],
        out_specs=pl.BlockSpec((C, H, W), lambda py: (0, 0, 0)),
        out_shape=jax.ShapeDtypeStruct((C, H, W), jnp.float32),
        scratch_shapes=[pltpu.VMEM((OH, C * KH, W + 2 * PAD), jnp.float32)],
    )(g3, s_mat)


def kernel(image, keys1, values2, mapping_table):
    img = jnp.transpose(image, (2, 0, 1))
    padded = jnp.pad(img, ((0, 0), (PAD, PAD), (PAD, PAD)))
    ri = jnp.stack([padded[:, i:i + OH, :] for i in range(KH)], axis=0)
    ri = jnp.transpose(ri, (2, 1, 0, 3)).reshape(OH, C * KH, W + 2 * PAD)
    pt = jnp.stack([ri[:, :, j:j + OW] for j in range(KW)], axis=0)
    pt = jnp.transpose(pt, (0, 2, 1, 3)).reshape(D, L)
    pt = jnp.pad(pt, ((0, 0), (0, L_PAD - L)))

    keys1r = keys1.reshape(K1, C * KH, KW).swapaxes(1, 2).reshape(K1, D)
    pat_ids = _nearest_pattern_ids(pt, keys1r, mapping_table.reshape(K1, 1))

    v3 = values2.reshape(K2, C * KH, KW).swapaxes(1, 2).reshape(K2, D)
    g = _sc_gather(pat_ids, v3)

    g3 = g[:L].reshape(OH, OW * KW, C * KH)
    out_chw = _fold_normalize(g3, _fold_matrix())
    return jnp.transpose(out_chw, (1, 2, 0))


# speedup vs baseline: 70.3968x; 1.0014x over previous
import functools

import jax
import jax.numpy as jnp
import numpy as np
from jax import lax
from jax.experimental import pallas as pl
from jax.experimental.pallas import tpu as pltpu
from jax.experimental.pallas import tpu_sc as plsc

KH, KW, PAD = 16, 16, 10
H, W, C = 128, 128, 3
D = C * KH * KW
K1, K2 = 2048, 1024
OH = H + 2 * PAD - KH + 1
OW = W + 2 * PAD - KW + 1
L = OH * OW
L_PAD = 17920
BLK_A = 512

NC, NS = 2, 16
NW = NC * NS
BPW = L_PAD // NW
CHUNK = 56
NCHUNK = BPW // CHUNK


def _nn_body(q_ref, k_ref, m_ref, ids_ref):
    q = q_ref[...]
    k = k_ref[...]
    dot = lax.dot_general(k, q, (((1,), (0,)), ((), ())),
                          preferred_element_type=jnp.float32)
    qsq = jnp.sum(q * q, axis=0, keepdims=True)
    ksq = jnp.sum(k * k, axis=1, keepdims=True)
    d2 = qsq - 2.0 * dot + ksq
    mn = jnp.min(d2, axis=0, keepdims=True)
    iota = lax.broadcasted_iota(jnp.int32, (K1, BLK_A), 0)
    ids = jnp.min(jnp.where(d2 == mn, iota, K1), axis=0)
    pat = jnp.sum(jnp.where(iota == ids[None, :], m_ref[...], 0), axis=0)
    ids_ref[0, 0, :] = pat


def _nearest_pattern_ids(patches_t, keys1r, mapping_col):
    grid = L_PAD // BLK_A
    out = pl.pallas_call(
        _nn_body,
        grid=(grid,),
        in_specs=[
            pl.BlockSpec((D, BLK_A), lambda m: (0, m)),
            pl.BlockSpec((K1, D), lambda m: (0, 0)),
            pl.BlockSpec((K1, 1), lambda m: (0, 0)),
        ],
        out_specs=pl.BlockSpec((1, 1, BLK_A), lambda m: (m, 0, 0)),
        out_shape=jax.ShapeDtypeStruct((grid, 1, BLK_A), jnp.int32),
    )(patches_t, keys1r, mapping_col)
    return out.reshape(L_PAD)


def _sc_gather(pat_ids, table):
    mesh = plsc.VectorSubcoreMesh(core_axis_name="c", subcore_axis_name="s")
    ids3 = pat_ids.reshape(NW, NCHUNK, CHUNK)

    @functools.partial(
        pl.kernel,
        mesh=mesh,
        out_type=jax.ShapeDtypeStruct((NW, NCHUNK, CHUNK, D), jnp.float32),
        scratch_types=[
            pltpu.VMEM((NCHUNK, CHUNK), jnp.int32),
            pltpu.VMEM((CHUNK, D), jnp.float32),
            pltpu.VMEM((CHUNK, D), jnp.float32),
            pltpu.SemaphoreType.DMA,
            pltpu.SemaphoreType.DMA,
        ],
    )
    def k(ids_hbm, tab_hbm, out_hbm, idx_v, rows0, rows1, sem0, sem1):
        wid = lax.axis_index("s") * NC + lax.axis_index("c")
        pltpu.sync_copy(ids_hbm.at[wid], idx_v)

        bufs, sems = (rows0, rows1), (sem0, sem1)
        pend = pltpu.async_copy(tab_hbm.at[idx_v.at[0]], bufs[0], sems[0])
        for ci in range(NCHUNK):
            nxt = None
            if ci + 1 < NCHUNK:
                nxt = pltpu.async_copy(
                    tab_hbm.at[idx_v.at[ci + 1]],
                    bufs[(ci + 1) % 2], sems[(ci + 1) % 2])
            pend.wait()
            pltpu.sync_copy(bufs[ci % 2], out_hbm.at[wid, ci])
            pend = nxt

    return k(ids3, table).reshape(L_PAD, D)


def _fold_matrix():
    s = np.zeros((OW * KW, W + 2 * PAD), np.float32)
    for px in range(OW):
        for j in range(KW):
            s[px * KW + j, px + j] = 1.0
    return jnp.asarray(s)


def _fold_body(g_ref, s_ref, out_ref, acc_ref):
    py = pl.program_id(0)
    gp = g_ref[0]
    r = lax.dot_general(gp, s_ref[...], (((0,), (0,)), ((), ())),
                        preferred_element_type=jnp.float32)
    acc_ref[py] = r

    @pl.when(py == OH - 1)
    def _():
        out_ref[...] = jnp.zeros((C, H, W), jnp.float32)
        for c in range(C):
            for i in range(KH):
                ylo = max(0, i - PAD)
                yhi = min(H - 1, H - KH + PAD + i)
                n = yhi - ylo + 1
                plo = ylo + PAD - i
                out_ref[c, ylo:ylo + n, :] += (
                    acc_ref[plo:plo + n, c * KH + i, PAD:PAD + W])
        m = jnp.max(out_ref[...])
        out_ref[...] = out_ref[...] / m


def _fold_normalize(g3, s_mat):
    return pl.pallas_call(
        _fold_body,
        grid=(OH,),
        in_specs=[
            pl.BlockSpec((1, OW * KW, C * KH), lambda py: (py, 0, 0)),
            pl.BlockSpec((OW * KW, W + 2 * PAD), lambda py: (0, 0)),
        ],
        out_specs=pl.BlockSpec((C, H, W), lambda py: (0, 0, 0)),
        out_shape=jax.ShapeDtypeStruct((C, H, W), jnp.float32),
        scratch_shapes=[pltpu.VMEM((OH, C * KH, W + 2 * PAD), jnp.float32)],
    )(g3, s_mat)


def kernel(image, keys1, values2, mapping_table):
    img = jnp.transpose(image, (2, 0, 1))
    padded = jnp.pad(img, ((0, 0), (PAD, PAD), (PAD, PAD)))
    ri = jnp.stack([padded[:, i:i + OH, :] for i in range(KH)], axis=0)
    ri = jnp.transpose(ri, (2, 1, 0, 3)).reshape(OH, C * KH, W + 2 * PAD)
    pt = jnp.stack([ri[:, :, j:j + OW] for j in range(KW)], axis=0)
    pt = jnp.transpose(pt, (0, 2, 1, 3)).reshape(D, L)
    pt = jnp.pad(pt, ((0, 0), (0, L_PAD - L)))

    keys1r = keys1.reshape(K1, C * KH, KW).swapaxes(1, 2).reshape(K1, D)
    pat_ids = _nearest_pattern_ids(pt, keys1r, mapping_table.reshape(K1, 1))

    v3 = values2.reshape(K2, C * KH, KW).swapaxes(1, 2).reshape(K2, D)
    g = _sc_gather(pat_ids, v3)

    g3 = g[:L].reshape(OH, OW * KW, C * KH)
    out_chw = _fold_normalize(g3, _fold_matrix())
    return jnp.transpose(out_chw, (1, 2, 0))
